# baseline (speedup 1.0000x reference)
import functools

import jax
import jax.numpy as jnp
import numpy as np
from jax import lax
from jax.experimental import pallas as pl
from jax.experimental.pallas import tpu as pltpu
from jax.experimental.pallas import tpu_sc as plsc

N = 10000
E = 320000
F = 128
D = 64
SCALE = 0.125
ACC_W = 144

NTILE = 32
EPT = 10240
EP = NTILE * EPT
NBLK = 80
BLK = 128
NTBL = 10240


def _sc_mesh():
    return plsc.VectorSubcoreMesh(core_axis_name="c", subcore_axis_name="s")


_SC_PARAMS = pltpu.CompilerParams(needs_layout_passes=False)


def _k_deg(dsts):
    @functools.partial(
        pl.kernel,
        out_type=jax.ShapeDtypeStruct((NTILE, NTBL), jnp.float32),
        mesh=_sc_mesh(),
        compiler_params=_SC_PARAMS,
        scratch_types=[
            pltpu.VMEM((NBLK, BLK), jnp.int32),
            pltpu.VMEM((NTBL,), jnp.float32),
        ],
    )
    def body(dst_hbm, out_hbm, dstv, degv):
        c = lax.axis_index("c")
        s = lax.axis_index("s")
        wid = c * 16 + s
        pltpu.sync_copy(dst_hbm.at[wid], dstv)

        def zero(i, carry):
            degv[pl.ds(i * 16, 16)] = jnp.zeros((16,), jnp.float32)
            return carry

        lax.fori_loop(0, NTBL // 16, zero, 0)
        ones = jnp.ones((16,), jnp.float32)

        def scat(i, carry):
            b = i // 8
            k = i % 8
            idx = dstv[b, pl.ds(k * 16, 16)]
            plsc.addupdate_scatter(degv, [idx], ones)
            return carry

        lax.fori_loop(0, EPT // 16, scat, 0)
        pltpu.sync_copy(degv, out_hbm.at[wid])

    return body(dsts)


def _dred_body(d_ref, out_ref):
    out_ref[...] = jnp.sum(d_ref[...], axis=0)[:, None]


def _dred(deg32):
    return pl.pallas_call(
        _dred_body,
        grid=(8,),
        in_specs=[pl.BlockSpec((NTILE, NTBL // 8), lambda i: (0, i))],
        out_specs=pl.BlockSpec((NTBL // 8, 1), lambda i: (i, 0)),
        out_shape=jax.ShapeDtypeStruct((NTBL, 1), jnp.float32),
    )(deg32)


GBLK = 64


def _k_gcn(xws, srcp, dsts):
    @functools.partial(
        pl.kernel,
        out_type=jax.ShapeDtypeStruct((2, NTBL, F), jnp.float32),
        mesh=_sc_mesh(),
        compiler_params=_SC_PARAMS,
        scratch_types=[
            pltpu.VMEM((8, GBLK), jnp.int32),
            pltpu.VMEM((8, GBLK), jnp.int32),
            pltpu.VMEM((GBLK, F), jnp.float32),
            pltpu.VMEM((GBLK, F), jnp.float32),
            pltpu.VMEM_SHARED((NTBL, F), jnp.float32),
            pltpu.SemaphoreType.DMA,
            pltpu.SemaphoreType.DMA,
            pltpu.SemaphoreType.DMA,
            pltpu.SemaphoreType.DMA,
        ],
    )
    def body(xws_hbm, src_hbm, dst_hbm, out_hbm, srcv, dstv, bufa, bufb,
             acc_s, sga, sgb, ssa, ssb):
        c = lax.axis_index("c")
        s = lax.axis_index("s")
        wid = c * 16 + s

        def zrows(i, carry):
            r = i // 8
            k = i % 8
            bufa[r, pl.ds(k * 16, 16)] = jnp.zeros((16,), jnp.float32)
            return carry

        lax.fori_loop(0, GBLK * 8, zrows, 0)

        def zacc(j, carry):
            pltpu.sync_copy(bufa, acc_s.at[pl.ds(s * 640 + j * GBLK, GBLK)])
            return carry

        lax.fori_loop(0, 640 // GBLK, zacc, 0)
        plsc.subcore_barrier()

        def grp(g, carry):
            pltpu.sync_copy(src_hbm.at[wid, pl.ds(g * 8, 8)], srcv)
            pltpu.sync_copy(dst_hbm.at[wid, pl.ds(g * 8, 8)], dstv)

            def blk2(p, carry2):
                cga = pltpu.async_copy(xws_hbm.at[srcv.at[2 * p]], bufa, sga)
                cgb = pltpu.async_copy(xws_hbm.at[srcv.at[2 * p + 1]], bufb,
                                       sgb)
                cga.wait()
                csa = pltpu.async_copy(bufa, acc_s.at[dstv.at[2 * p]], ssa,
                                       add=True)
                cgb.wait()
                csb = pltpu.async_copy(bufb, acc_s.at[dstv.at[2 * p + 1]], ssb,
                                       add=True)
                csa.wait()
                csb.wait()
                return carry2

            lax.fori_loop(0, 4, blk2, 0)
            return carry

        lax.fori_loop(0, EPT // (8 * GBLK), grp, 0)
        plsc.subcore_barrier()

        def outc(j, carry):
            off = s * 640 + j * GBLK
            pltpu.sync_copy(acc_s.at[pl.ds(off, GBLK)], bufa)
            pltpu.sync_copy(bufa, out_hbm.at[c, pl.ds(off, GBLK)])
            return carry

        lax.fori_loop(0, 640 // GBLK, outc, 0)

    return body(xws, srcp, dsts)


ABLK = 32
AGRP = 8
NGRP = EPT // (ABLK * AGRP)


def _k_att(qq, kv, srcp, dstg, dsts):
    @functools.partial(
        pl.kernel,
        out_type=(jax.ShapeDtypeStruct((2, NTBL, F), jnp.float32),
                  jax.ShapeDtypeStruct((NTILE, 2 * NTBL), jnp.float32)),
        mesh=_sc_mesh(),
        compiler_params=_SC_PARAMS,
        scratch_types=[
            pltpu.VMEM((AGRP, ABLK), jnp.int32),
            pltpu.VMEM((AGRP, ABLK), jnp.int32),
            pltpu.VMEM((AGRP, ABLK), jnp.int32),
            pltpu.VMEM((ABLK, F), jnp.float32),
            pltpu.VMEM((ABLK, 2 * F), jnp.float32),
            pltpu.VMEM((ABLK, F), jnp.float32),
            pltpu.VMEM((ABLK,), jnp.float32),
            pltpu.VMEM((ABLK,), jnp.float32),
            pltpu.VMEM((2 * NTBL,), jnp.float32),
            pltpu.VMEM_SHARED((NTBL, F), jnp.float32),
            pltpu.SemaphoreType.DMA,
            pltpu.SemaphoreType.DMA,
        ],
    )
    def body(qq_hbm, kv_hbm, src_hbm, dstg_hbm, dsts_hbm, out_hbm, den_hbm,
             srcv, dgv, dsv, qd, kvs, rows, dbm, dbl, dtab, att_s, sem1, sem2):
        c = lax.axis_index("c")
        s = lax.axis_index("s")
        wid = c * 16 + s
        zero16 = jnp.zeros((16,), jnp.float32)
        lanes = lax.iota(jnp.int32, 16)

        def zrows(i, carry):
            r = i // 8
            k = i % 8
            rows[r, pl.ds(k * 16, 16)] = zero16
            return carry

        lax.fori_loop(0, ABLK * 8, zrows, 0)

        def zden(i, carry):
            dtab[pl.ds(i * 16, 16)] = zero16
            return carry

        lax.fori_loop(0, 2 * NTBL // 16, zden, 0)

        def zacc(j, carry):
            pltpu.sync_copy(rows, att_s.at[pl.ds(s * 640 + j * ABLK, ABLK)])
            return carry

        lax.fori_loop(0, 640 // ABLK, zacc, 0)
        plsc.subcore_barrier()

        mask0 = lanes == 0

        def run_edges():
          @plsc.parallel_loop(0, ABLK, 1, unroll=4)
          def edge(e):
            amu = (qd[e, pl.ds(0, 16)] * kvs[e, pl.ds(0, 16)]
                   + qd[e, pl.ds(16, 16)] * kvs[e, pl.ds(16, 16)]
                   + qd[e, pl.ds(32, 16)] * kvs[e, pl.ds(32, 16)]
                   + qd[e, pl.ds(48, 16)] * kvs[e, pl.ds(48, 16)])
            als = (qd[e, pl.ds(64, 16)] * kvs[e, pl.ds(64, 16)]
                   + qd[e, pl.ds(80, 16)] * kvs[e, pl.ds(80, 16)]
                   + qd[e, pl.ds(96, 16)] * kvs[e, pl.ds(96, 16)]
                   + qd[e, pl.ds(112, 16)] * kvs[e, pl.ds(112, 16)])
            emu = jnp.exp(jnp.broadcast_to(jnp.sum(amu) * SCALE, (16,)))
            els = jnp.exp(jnp.broadcast_to(jnp.sum(als) * SCALE, (16,)))
            rows[e, pl.ds(0, 16)] = kvs[e, pl.ds(128, 16)] * emu
            rows[e, pl.ds(16, 16)] = kvs[e, pl.ds(144, 16)] * emu
            rows[e, pl.ds(32, 16)] = kvs[e, pl.ds(160, 16)] * emu
            rows[e, pl.ds(48, 16)] = kvs[e, pl.ds(176, 16)] * emu
            rows[e, pl.ds(64, 16)] = kvs[e, pl.ds(192, 16)] * els
            rows[e, pl.ds(80, 16)] = kvs[e, pl.ds(208, 16)] * els
            rows[e, pl.ds(96, 16)] = kvs[e, pl.ds(224, 16)] * els
            rows[e, pl.ds(112, 16)] = kvs[e, pl.ds(240, 16)] * els
            ei = jnp.full((16,), e, jnp.int32)
            plsc.store_scatter(dbm, [ei], emu, mask=mask0)
            plsc.store_scatter(dbl, [ei], els, mask=mask0)

        def make_dscat(b2):
          def dscat(t, carry):
            di = dsv[b2, pl.ds(t * 16, 16)]
            plsc.addupdate_scatter(dtab, [di * 2], dbm[pl.ds(t * 16, 16)])
            plsc.addupdate_scatter(dtab, [di * 2 + 1], dbl[pl.ds(t * 16, 16)])
            return carry
          return dscat

        def grp(g, carry):
            pltpu.sync_copy(src_hbm.at[wid, pl.ds(g * AGRP, AGRP)], srcv)
            pltpu.sync_copy(dstg_hbm.at[wid, pl.ds(g * AGRP, AGRP)], dgv)
            pltpu.sync_copy(dsts_hbm.at[wid, pl.ds(g * AGRP, AGRP)], dsv)

            def blk(b2, carry2):
                cp1 = pltpu.async_copy(qq_hbm.at[dgv.at[b2]], qd, sem1)
                cp2 = pltpu.async_copy(kv_hbm.at[srcv.at[b2]], kvs, sem2)
                cp1.wait()
                cp2.wait()
                run_edges()
                lax.fori_loop(0, ABLK // 16, make_dscat(b2), 0)
                pltpu.sync_copy(rows, att_s.at[dsv.at[b2]], add=True)
                return carry2

            lax.fori_loop(0, AGRP, blk, 0)
            return carry

        lax.fori_loop(0, NGRP, grp, 0)
        plsc.subcore_barrier()

        def outc(j, carry):
            off = s * 640 + j * ABLK
            pltpu.sync_copy(att_s.at[pl.ds(off, ABLK)], rows)
            pltpu.sync_copy(rows, out_hbm.at[c, pl.ds(off, ABLK)])
            return carry

        lax.fori_loop(0, 640 // ABLK, outc, 0)
        pltpu.sync_copy(dtab, den_hbm.at[wid])

    return body(qq, kv, srcp, dstg, dsts)


_BN = 2000


def _row_spec(w):
    return pl.BlockSpec((_BN, w), lambda i: (i, 0))


def _full_spec(h, w):
    return pl.BlockSpec((h, w), lambda i: (0, 0))


def _mm1_body(x_ref, w_ref, deg_ref, xws_ref, dis_ref):
    dis = jax.lax.rsqrt(jnp.maximum(deg_ref[...] + 1.0, 1.0))
    xw = jnp.dot(x_ref[...], w_ref[...], preferred_element_type=jnp.float32)
    xws_ref[...] = xw * dis
    dis_ref[...] = dis


def _mm1(x, w, deg):
    return pl.pallas_call(
        _mm1_body,
        grid=(N // _BN,),
        in_specs=[_row_spec(F), _full_spec(F, F), _row_spec(1)],
        out_specs=(_row_spec(F), _row_spec(1)),
        out_shape=(
            jax.ShapeDtypeStruct((N, F), jnp.float32),
            jax.ShapeDtypeStruct((N, 1), jnp.float32),
        ),
    )(x, w, deg)


def _mm2_body(accg_ref, xws_ref, dis_ref, b_ref, wcat_ref, bcat_ref,
              qq_ref, kv_ref, init_ref, dinit_ref, skip_ref):
    acc = accg_ref[0] + accg_ref[1] + xws_ref[...]
    gcn = acc * dis_ref[...] + b_ref[...]
    h = jnp.where(gcn >= 0.0, gcn, 0.01 * gcn)
    z = jnp.dot(h, wcat_ref[...], preferred_element_type=jnp.float32) + bcat_ref[...]
    q_mu, k_mu, v_mu, s_mu = z[:, 0:64], z[:, 64:128], z[:, 128:192], z[:, 192:256]
    q_ls, k_ls, v_ls, s_ls = z[:, 256:320], z[:, 320:384], z[:, 384:448], z[:, 448:512]
    qq_ref[...] = jnp.concatenate([q_mu, q_ls], axis=1)
    kv_ref[...] = jnp.concatenate([k_mu, k_ls, v_mu, v_ls], axis=1)
    e_mu = jnp.exp(jnp.sum(q_mu * k_mu, axis=1, keepdims=True) * SCALE)
    e_ls = jnp.exp(jnp.sum(q_ls * k_ls, axis=1, keepdims=True) * SCALE)
    init_ref[...] = jnp.concatenate([e_mu * v_mu, e_ls * v_ls], axis=1)
    dinit_ref[...] = jnp.concatenate([e_mu, e_ls], axis=1)
    skip_ref[...] = jnp.concatenate([s_mu, s_ls], axis=1)


def _mm2(accg, xws, dis, b_gcn, wcat, bcat):
    return pl.pallas_call(
        _mm2_body,
        grid=(N // _BN,),
        in_specs=[pl.BlockSpec((2, _BN, F), lambda i: (0, i, 0)),
                  _row_spec(F), _row_spec(1), _full_spec(1, F),
                  _full_spec(F, 512), _full_spec(1, 512)],
        out_specs=(_row_spec(2 * D), _row_spec(4 * D), _row_spec(F),
                   _row_spec(2), _row_spec(2 * D)),
        out_shape=(
            jax.ShapeDtypeStruct((N, 2 * D), jnp.float32),
            jax.ShapeDtypeStruct((N, 4 * D), jnp.float32),
            jax.ShapeDtypeStruct((N, F), jnp.float32),
            jax.ShapeDtypeStruct((N, 2), jnp.float32),
            jax.ShapeDtypeStruct((N, 2 * D), jnp.float32),
        ),
    )(accg, xws, dis, b_gcn, wcat, bcat)


def _dred2_body(d_ref, out_ref):
    out_ref[...] = jnp.sum(d_ref[...], axis=0)


def _dred2(den32):
    return pl.pallas_call(
        _dred2_body,
        grid=(8,),
        in_specs=[pl.BlockSpec((NTILE, NTBL // 8, 2), lambda i: (0, i, 0))],
        out_specs=pl.BlockSpec((NTBL // 8, 2), lambda i: (i, 0)),
        out_shape=jax.ShapeDtypeStruct((NTBL, 2), jnp.float32),
    )(den32.reshape(NTILE, NTBL, 2))


def _out_body(attg_ref, den_ref, init_ref, dinit_ref, skip_ref, mu_ref, ls_ref):
    att = attg_ref[0] + attg_ref[1] + init_ref[...]
    den = den_ref[...] + dinit_ref[...]
    mu = att[:, 0:64] / (den[:, 0:1] + 1e-16) + skip_ref[:, 0:64]
    ls = att[:, 64:128] / (den[:, 1:2] + 1e-16) + skip_ref[:, 64:128]
    mu_ref[...] = mu
    ls_ref[...] = jnp.minimum(ls, 10.0)


def _out(attg, den, init, dinit, skip):
    return pl.pallas_call(
        _out_body,
        grid=(N // _BN,),
        in_specs=[pl.BlockSpec((2, _BN, F), lambda i: (0, i, 0)),
                  _row_spec(2), _row_spec(F), _row_spec(2), _row_spec(2 * D)],
        out_specs=(_row_spec(D), _row_spec(D)),
        out_shape=(
            jax.ShapeDtypeStruct((N, D), jnp.float32),
            jax.ShapeDtypeStruct((N, D), jnp.float32),
        ),
    )(attg, den, init, dinit, skip)


def kernel(x, edge_index, W_gcn, b_gcn, Wq_mu, bq_mu, Wk_mu, bk_mu, Wv_mu, bv_mu,
           Ws_mu, bs_mu, Wq_ls, bq_ls, Wk_ls, bk_ls, Wv_ls, bv_ls, Ws_ls, bs_ls):
    src = edge_index[0]
    dst = edge_index[1]

    pad = EP - E
    srcp = jnp.concatenate([src, jnp.zeros((pad,), jnp.int32)])
    srcp = srcp.reshape(NTILE, NBLK, BLK)
    dsts = jnp.concatenate([dst, jnp.full((pad,), N, jnp.int32)])
    dsts = dsts.reshape(NTILE, NBLK, BLK)
    dstg = jnp.concatenate([dst, jnp.zeros((pad,), jnp.int32)])
    dstg = dstg.reshape(NTILE, NBLK, BLK)

    deg = _dred(_k_deg(dsts))
    xws, dis = _mm1(x, W_gcn, deg[:N])

    gshape = (NTILE, EPT // GBLK, GBLK)
    accg = _k_gcn(xws, srcp.reshape(gshape), dsts.reshape(gshape))

    wcat = jnp.concatenate(
        [Wq_mu, Wk_mu, Wv_mu, Ws_mu, Wq_ls, Wk_ls, Wv_ls, Ws_ls], axis=1)
    bcat = jnp.concatenate(
        [bq_mu, bk_mu, bv_mu, bs_mu, bq_ls, bk_ls, bv_ls, bs_ls])[None, :]
    qq, kv, att_init, den_init, skip = _mm2(accg, xws, dis, b_gcn[None, :],
                                            wcat, bcat)

    ashape = (NTILE, EPT // ABLK, ABLK)
    attg, den32 = _k_att(qq, kv, srcp.reshape(ashape), dstg.reshape(ashape),
                         dsts.reshape(ashape))
    den = _dred2(den32)

    return _out(attg, den[:N], att_init, den_init, skip)


# speedup vs baseline: 1.0375x; 1.0375x over previous
import functools

import jax
import jax.numpy as jnp
import numpy as np
from jax import lax
from jax.experimental import pallas as pl
from jax.experimental.pallas import tpu as pltpu
from jax.experimental.pallas import tpu_sc as plsc

N = 10000
E = 320000
F = 128
D = 64
SCALE = 0.125
ACC_W = 144

NTILE = 32
EPT = 10240
EP = NTILE * EPT
NBLK = 80
BLK = 128
NTBL = 10240


def _sc_mesh():
    return plsc.VectorSubcoreMesh(core_axis_name="c", subcore_axis_name="s")


_SC_PARAMS = pltpu.CompilerParams(needs_layout_passes=False)


def _k_deg(dsts):
    @functools.partial(
        pl.kernel,
        out_type=jax.ShapeDtypeStruct((NTILE, NTBL), jnp.float32),
        mesh=_sc_mesh(),
        compiler_params=_SC_PARAMS,
        scratch_types=[
            pltpu.VMEM((NBLK, BLK), jnp.int32),
            pltpu.VMEM((NTBL,), jnp.float32),
        ],
    )
    def body(dst_hbm, out_hbm, dstv, degv):
        c = lax.axis_index("c")
        s = lax.axis_index("s")
        wid = c * 16 + s
        pltpu.sync_copy(dst_hbm.at[wid], dstv)

        def zero(i, carry):
            degv[pl.ds(i * 16, 16)] = jnp.zeros((16,), jnp.float32)
            return carry

        lax.fori_loop(0, NTBL // 16, zero, 0)
        ones = jnp.ones((16,), jnp.float32)

        def scat(i, carry):
            b = i // 8
            k = i % 8
            idx = dstv[b, pl.ds(k * 16, 16)]
            plsc.addupdate_scatter(degv, [idx], ones)
            return carry

        lax.fori_loop(0, EPT // 16, scat, 0)
        pltpu.sync_copy(degv, out_hbm.at[wid])

    return body(dsts)


def _dred_body(d_ref, out_ref):
    out_ref[...] = jnp.sum(d_ref[...], axis=0)[:, None]


def _dred(deg32):
    return pl.pallas_call(
        _dred_body,
        grid=(8,),
        in_specs=[pl.BlockSpec((NTILE, NTBL // 8), lambda i: (0, i))],
        out_specs=pl.BlockSpec((NTBL // 8, 1), lambda i: (i, 0)),
        out_shape=jax.ShapeDtypeStruct((NTBL, 1), jnp.float32),
    )(deg32)


GBLK = 128


def _k_gcn(xws, srcp, dsts):
    @functools.partial(
        pl.kernel,
        out_type=jax.ShapeDtypeStruct((2, NTBL, F), jnp.float32),
        mesh=_sc_mesh(),
        compiler_params=_SC_PARAMS,
        scratch_types=[
            pltpu.VMEM((8, GBLK), jnp.int32),
            pltpu.VMEM((8, GBLK), jnp.int32),
            pltpu.VMEM((GBLK, F), jnp.float32),
            pltpu.VMEM((GBLK, F), jnp.float32),
            pltpu.VMEM_SHARED((NTBL, F), jnp.float32),
            pltpu.SemaphoreType.DMA,
            pltpu.SemaphoreType.DMA,
            pltpu.SemaphoreType.DMA,
            pltpu.SemaphoreType.DMA,
        ],
    )
    def body(xws_hbm, src_hbm, dst_hbm, out_hbm, srcv, dstv, bufa, bufb,
             acc_s, sga, sgb, ssa, ssb):
        c = lax.axis_index("c")
        s = lax.axis_index("s")
        wid = c * 16 + s

        def zrows(i, carry):
            r = i // 8
            k = i % 8
            bufa[r, pl.ds(k * 16, 16)] = jnp.zeros((16,), jnp.float32)
            return carry

        lax.fori_loop(0, GBLK * 8, zrows, 0)

        def zacc(j, carry):
            pltpu.sync_copy(bufa, acc_s.at[pl.ds(s * 640 + j * GBLK, GBLK)])
            return carry

        lax.fori_loop(0, 640 // GBLK, zacc, 0)
        plsc.subcore_barrier()

        def grp(g, carry):
            pltpu.sync_copy(src_hbm.at[wid, pl.ds(g * 8, 8)], srcv)
            pltpu.sync_copy(dst_hbm.at[wid, pl.ds(g * 8, 8)], dstv)

            def blk2(p, carry2):
                cga = pltpu.async_copy(xws_hbm.at[srcv.at[2 * p]], bufa, sga)
                cgb = pltpu.async_copy(xws_hbm.at[srcv.at[2 * p + 1]], bufb,
                                       sgb)
                cga.wait()
                csa = pltpu.async_copy(bufa, acc_s.at[dstv.at[2 * p]], ssa,
                                       add=True)
                cgb.wait()
                csb = pltpu.async_copy(bufb, acc_s.at[dstv.at[2 * p + 1]], ssb,
                                       add=True)
                csa.wait()
                csb.wait()
                return carry2

            lax.fori_loop(0, 4, blk2, 0)
            return carry

        lax.fori_loop(0, EPT // (8 * GBLK), grp, 0)
        plsc.subcore_barrier()

        def outc(j, carry):
            off = s * 640 + j * GBLK
            pltpu.sync_copy(acc_s.at[pl.ds(off, GBLK)], bufa)
            pltpu.sync_copy(bufa, out_hbm.at[c, pl.ds(off, GBLK)])
            return carry

        lax.fori_loop(0, 640 // GBLK, outc, 0)

    return body(xws, srcp, dsts)


ABLK = 32
AGRP = 8
NGRP = EPT // (ABLK * AGRP)


def _k_att(qq, kv, srcp, dstg, dsts):
    @functools.partial(
        pl.kernel,
        out_type=(jax.ShapeDtypeStruct((2, NTBL, F), jnp.float32),
                  jax.ShapeDtypeStruct((NTILE, 2 * NTBL), jnp.float32)),
        mesh=_sc_mesh(),
        compiler_params=_SC_PARAMS,
        scratch_types=[
            pltpu.VMEM((AGRP, ABLK), jnp.int32),
            pltpu.VMEM((AGRP, ABLK), jnp.int32),
            pltpu.VMEM((AGRP, ABLK), jnp.int32),
            pltpu.VMEM((ABLK, F), jnp.float32),
            pltpu.VMEM((ABLK, 2 * F), jnp.float32),
            pltpu.VMEM((ABLK, F), jnp.float32),
            pltpu.VMEM((ABLK,), jnp.float32),
            pltpu.VMEM((ABLK,), jnp.float32),
            pltpu.VMEM((2 * NTBL,), jnp.float32),
            pltpu.VMEM_SHARED((NTBL, F), jnp.float32),
            pltpu.SemaphoreType.DMA,
            pltpu.SemaphoreType.DMA,
        ],
    )
    def body(qq_hbm, kv_hbm, src_hbm, dstg_hbm, dsts_hbm, out_hbm, den_hbm,
             srcv, dgv, dsv, qd, kvs, rows, dbm, dbl, dtab, att_s, sem1, sem2):
        c = lax.axis_index("c")
        s = lax.axis_index("s")
        wid = c * 16 + s
        zero16 = jnp.zeros((16,), jnp.float32)
        lanes = lax.iota(jnp.int32, 16)

        def zrows(i, carry):
            r = i // 8
            k = i % 8
            rows[r, pl.ds(k * 16, 16)] = zero16
            return carry

        lax.fori_loop(0, ABLK * 8, zrows, 0)

        def zden(i, carry):
            dtab[pl.ds(i * 16, 16)] = zero16
            return carry

        lax.fori_loop(0, 2 * NTBL // 16, zden, 0)

        def zacc(j, carry):
            pltpu.sync_copy(rows, att_s.at[pl.ds(s * 640 + j * ABLK, ABLK)])
            return carry

        lax.fori_loop(0, 640 // ABLK, zacc, 0)
        plsc.subcore_barrier()

        mask0 = lanes == 0

        def run_edges():
          @plsc.parallel_loop(0, ABLK, 1, unroll=4)
          def edge(e):
            amu = (qd[e, pl.ds(0, 16)] * kvs[e, pl.ds(0, 16)]
                   + qd[e, pl.ds(16, 16)] * kvs[e, pl.ds(16, 16)]
                   + qd[e, pl.ds(32, 16)] * kvs[e, pl.ds(32, 16)]
                   + qd[e, pl.ds(48, 16)] * kvs[e, pl.ds(48, 16)])
            als = (qd[e, pl.ds(64, 16)] * kvs[e, pl.ds(64, 16)]
                   + qd[e, pl.ds(80, 16)] * kvs[e, pl.ds(80, 16)]
                   + qd[e, pl.ds(96, 16)] * kvs[e, pl.ds(96, 16)]
                   + qd[e, pl.ds(112, 16)] * kvs[e, pl.ds(112, 16)])
            emu = jnp.exp(jnp.broadcast_to(jnp.sum(amu) * SCALE, (16,)))
            els = jnp.exp(jnp.broadcast_to(jnp.sum(als) * SCALE, (16,)))
            rows[e, pl.ds(0, 16)] = kvs[e, pl.ds(128, 16)] * emu
            rows[e, pl.ds(16, 16)] = kvs[e, pl.ds(144, 16)] * emu
            rows[e, pl.ds(32, 16)] = kvs[e, pl.ds(160, 16)] * emu
            rows[e, pl.ds(48, 16)] = kvs[e, pl.ds(176, 16)] * emu
            rows[e, pl.ds(64, 16)] = kvs[e, pl.ds(192, 16)] * els
            rows[e, pl.ds(80, 16)] = kvs[e, pl.ds(208, 16)] * els
            rows[e, pl.ds(96, 16)] = kvs[e, pl.ds(224, 16)] * els
            rows[e, pl.ds(112, 16)] = kvs[e, pl.ds(240, 16)] * els
            ei = jnp.full((16,), e, jnp.int32)
            plsc.store_scatter(dbm, [ei], emu, mask=mask0)
            plsc.store_scatter(dbl, [ei], els, mask=mask0)

        def make_dscat(b2):
          def dscat(t, carry):
            di = dsv[b2, pl.ds(t * 16, 16)]
            plsc.addupdate_scatter(dtab, [di * 2], dbm[pl.ds(t * 16, 16)])
            plsc.addupdate_scatter(dtab, [di * 2 + 1], dbl[pl.ds(t * 16, 16)])
            return carry
          return dscat

        def grp(g, carry):
            pltpu.sync_copy(src_hbm.at[wid, pl.ds(g * AGRP, AGRP)], srcv)
            pltpu.sync_copy(dstg_hbm.at[wid, pl.ds(g * AGRP, AGRP)], dgv)
            pltpu.sync_copy(dsts_hbm.at[wid, pl.ds(g * AGRP, AGRP)], dsv)

            def blk(b2, carry2):
                cp1 = pltpu.async_copy(qq_hbm.at[dgv.at[b2]], qd, sem1)
                cp2 = pltpu.async_copy(kv_hbm.at[srcv.at[b2]], kvs, sem2)
                cp1.wait()
                cp2.wait()
                run_edges()
                lax.fori_loop(0, ABLK // 16, make_dscat(b2), 0)
                pltpu.sync_copy(rows, att_s.at[dsv.at[b2]], add=True)
                return carry2

            lax.fori_loop(0, AGRP, blk, 0)
            return carry

        lax.fori_loop(0, NGRP, grp, 0)
        plsc.subcore_barrier()

        def outc(j, carry):
            off = s * 640 + j * ABLK
            pltpu.sync_copy(att_s.at[pl.ds(off, ABLK)], rows)
            pltpu.sync_copy(rows, out_hbm.at[c, pl.ds(off, ABLK)])
            return carry

        lax.fori_loop(0, 640 // ABLK, outc, 0)
        pltpu.sync_copy(dtab, den_hbm.at[wid])

    return body(qq, kv, srcp, dstg, dsts)


_BN = 2000


def _row_spec(w):
    return pl.BlockSpec((_BN, w), lambda i: (i, 0))


def _full_spec(h, w):
    return pl.BlockSpec((h, w), lambda i: (0, 0))


def _mm1_body(x_ref, w_ref, deg_ref, xws_ref, dis_ref):
    dis = jax.lax.rsqrt(jnp.maximum(deg_ref[...] + 1.0, 1.0))
    xw = jnp.dot(x_ref[...], w_ref[...], preferred_element_type=jnp.float32)
    xws_ref[...] = xw * dis
    dis_ref[...] = dis


def _mm1(x, w, deg):
    return pl.pallas_call(
        _mm1_body,
        grid=(N // _BN,),
        in_specs=[_row_spec(F), _full_spec(F, F), _row_spec(1)],
        out_specs=(_row_spec(F), _row_spec(1)),
        out_shape=(
            jax.ShapeDtypeStruct((N, F), jnp.float32),
            jax.ShapeDtypeStruct((N, 1), jnp.float32),
        ),
    )(x, w, deg)


def _mm2_body(accg_ref, xws_ref, dis_ref, b_ref, wcat_ref, bcat_ref,
              qq_ref, kv_ref, init_ref, dinit_ref, skip_ref):
    acc = accg_ref[0] + accg_ref[1] + xws_ref[...]
    gcn = acc * dis_ref[...] + b_ref[...]
    h = jnp.where(gcn >= 0.0, gcn, 0.01 * gcn)
    z = jnp.dot(h, wcat_ref[...], preferred_element_type=jnp.float32) + bcat_ref[...]
    q_mu, k_mu, v_mu, s_mu = z[:, 0:64], z[:, 64:128], z[:, 128:192], z[:, 192:256]
    q_ls, k_ls, v_ls, s_ls = z[:, 256:320], z[:, 320:384], z[:, 384:448], z[:, 448:512]
    qq_ref[...] = jnp.concatenate([q_mu, q_ls], axis=1)
    kv_ref[...] = jnp.concatenate([k_mu, k_ls, v_mu, v_ls], axis=1)
    e_mu = jnp.exp(jnp.sum(q_mu * k_mu, axis=1, keepdims=True) * SCALE)
    e_ls = jnp.exp(jnp.sum(q_ls * k_ls, axis=1, keepdims=True) * SCALE)
    init_ref[...] = jnp.concatenate([e_mu * v_mu, e_ls * v_ls], axis=1)
    dinit_ref[...] = jnp.concatenate([e_mu, e_ls], axis=1)
    skip_ref[...] = jnp.concatenate([s_mu, s_ls], axis=1)


def _mm2(accg, xws, dis, b_gcn, wcat, bcat):
    return pl.pallas_call(
        _mm2_body,
        grid=(N // _BN,),
        in_specs=[pl.BlockSpec((2, _BN, F), lambda i: (0, i, 0)),
                  _row_spec(F), _row_spec(1), _full_spec(1, F),
                  _full_spec(F, 512), _full_spec(1, 512)],
        out_specs=(_row_spec(2 * D), _row_spec(4 * D), _row_spec(F),
                   _row_spec(2), _row_spec(2 * D)),
        out_shape=(
            jax.ShapeDtypeStruct((N, 2 * D), jnp.float32),
            jax.ShapeDtypeStruct((N, 4 * D), jnp.float32),
            jax.ShapeDtypeStruct((N, F), jnp.float32),
            jax.ShapeDtypeStruct((N, 2), jnp.float32),
            jax.ShapeDtypeStruct((N, 2 * D), jnp.float32),
        ),
    )(accg, xws, dis, b_gcn, wcat, bcat)


def _dred2_body(d_ref, out_ref):
    out_ref[...] = jnp.sum(d_ref[...], axis=0)


def _dred2(den32):
    return pl.pallas_call(
        _dred2_body,
        grid=(8,),
        in_specs=[pl.BlockSpec((NTILE, NTBL // 8, 2), lambda i: (0, i, 0))],
        out_specs=pl.BlockSpec((NTBL // 8, 2), lambda i: (i, 0)),
        out_shape=jax.ShapeDtypeStruct((NTBL, 2), jnp.float32),
    )(den32.reshape(NTILE, NTBL, 2))


def _out_body(attg_ref, den_ref, init_ref, dinit_ref, skip_ref, mu_ref, ls_ref):
    att = attg_ref[0] + attg_ref[1] + init_ref[...]
    den = den_ref[...] + dinit_ref[...]
    mu = att[:, 0:64] / (den[:, 0:1] + 1e-16) + skip_ref[:, 0:64]
    ls = att[:, 64:128] / (den[:, 1:2] + 1e-16) + skip_ref[:, 64:128]
    mu_ref[...] = mu
    ls_ref[...] = jnp.minimum(ls, 10.0)


def _out(attg, den, init, dinit, skip):
    return pl.pallas_call(
        _out_body,
        grid=(N // _BN,),
        in_specs=[pl.BlockSpec((2, _BN, F), lambda i: (0, i, 0)),
                  _row_spec(2), _row_spec(F), _row_spec(2), _row_spec(2 * D)],
        out_specs=(_row_spec(D), _row_spec(D)),
        out_shape=(
            jax.ShapeDtypeStruct((N, D), jnp.float32),
            jax.ShapeDtypeStruct((N, D), jnp.float32),
        ),
    )(attg, den, init, dinit, skip)


def kernel(x, edge_index, W_gcn, b_gcn, Wq_mu, bq_mu, Wk_mu, bk_mu, Wv_mu, bv_mu,
           Ws_mu, bs_mu, Wq_ls, bq_ls, Wk_ls, bk_ls, Wv_ls, bv_ls, Ws_ls, bs_ls):
    src = edge_index[0]
    dst = edge_index[1]

    pad = EP - E
    srcp = jnp.concatenate([src, jnp.zeros((pad,), jnp.int32)])
    srcp = srcp.reshape(NTILE, NBLK, BLK)
    dsts = jnp.concatenate([dst, jnp.full((pad,), N, jnp.int32)])
    dsts = dsts.reshape(NTILE, NBLK, BLK)
    dstg = jnp.concatenate([dst, jnp.zeros((pad,), jnp.int32)])
    dstg = dstg.reshape(NTILE, NBLK, BLK)

    deg = _dred(_k_deg(dsts))
    xws, dis = _mm1(x, W_gcn, deg[:N])

    gshape = (NTILE, EPT // GBLK, GBLK)
    accg = _k_gcn(xws, srcp.reshape(gshape), dsts.reshape(gshape))

    wcat = jnp.concatenate(
        [Wq_mu, Wk_mu, Wv_mu, Ws_mu, Wq_ls, Wk_ls, Wv_ls, Ws_ls], axis=1)
    bcat = jnp.concatenate(
        [bq_mu, bk_mu, bv_mu, bs_mu, bq_ls, bk_ls, bv_ls, bs_ls])[None, :]
    qq, kv, att_init, den_init, skip = _mm2(accg, xws, dis, b_gcn[None, :],
                                            wcat, bcat)

    ashape = (NTILE, EPT // ABLK, ABLK)
    attg, den32 = _k_att(qq, kv, srcp.reshape(ashape), dstg.reshape(ashape),
                         dsts.reshape(ashape))
    den = _dred2(den32)

    return _out(attg, den[:N], att_init, den_init, skip)


# speedup vs baseline: 1.0585x; 1.0202x over previous
import functools

import jax
import jax.numpy as jnp
import numpy as np
from jax import lax
from jax.experimental import pallas as pl
from jax.experimental.pallas import tpu as pltpu
from jax.experimental.pallas import tpu_sc as plsc

N = 10000
E = 320000
F = 128
D = 64
SCALE = 0.125
ACC_W = 144

NTILE = 32
EPT = 10240
EP = NTILE * EPT
NBLK = 80
BLK = 128
NTBL = 10240


def _sc_mesh():
    return plsc.VectorSubcoreMesh(core_axis_name="c", subcore_axis_name="s")


_SC_PARAMS = pltpu.CompilerParams(needs_layout_passes=False)


def _k_deg(dsts):
    @functools.partial(
        pl.kernel,
        out_type=jax.ShapeDtypeStruct((NTILE, NTBL), jnp.float32),
        mesh=_sc_mesh(),
        compiler_params=_SC_PARAMS,
        scratch_types=[
            pltpu.VMEM((NBLK, BLK), jnp.int32),
            pltpu.VMEM((NTBL,), jnp.float32),
        ],
    )
    def body(dst_hbm, out_hbm, dstv, degv):
        c = lax.axis_index("c")
        s = lax.axis_index("s")
        wid = c * 16 + s
        pltpu.sync_copy(dst_hbm.at[wid], dstv)

        def zero(i, carry):
            degv[pl.ds(i * 16, 16)] = jnp.zeros((16,), jnp.float32)
            return carry

        lax.fori_loop(0, NTBL // 16, zero, 0)
        ones = jnp.ones((16,), jnp.float32)

        def scat(i, carry):
            b = i // 8
            k = i % 8
            idx = dstv[b, pl.ds(k * 16, 16)]
            plsc.addupdate_scatter(degv, [idx], ones)
            return carry

        lax.fori_loop(0, EPT // 16, scat, 0)
        pltpu.sync_copy(degv, out_hbm.at[wid])

    return body(dsts)


def _dred_body(d_ref, out_ref):
    out_ref[...] = jnp.sum(d_ref[...], axis=0)[:, None]


def _dred(deg32):
    return pl.pallas_call(
        _dred_body,
        grid=(8,),
        in_specs=[pl.BlockSpec((NTILE, NTBL // 8), lambda i: (0, i))],
        out_specs=pl.BlockSpec((NTBL // 8, 1), lambda i: (i, 0)),
        out_shape=jax.ShapeDtypeStruct((NTBL, 1), jnp.float32),
    )(deg32)


GBLK = 128


def _k_gcn(xws, srcp, dsts):
    @functools.partial(
        pl.kernel,
        out_type=jax.ShapeDtypeStruct((2, NTBL, F), jnp.float32),
        mesh=_sc_mesh(),
        compiler_params=_SC_PARAMS,
        scratch_types=[
            pltpu.VMEM((8, GBLK), jnp.int32),
            pltpu.VMEM((8, GBLK), jnp.int32),
            pltpu.VMEM((GBLK, F), jnp.float32),
            pltpu.VMEM((GBLK, F), jnp.float32),
            pltpu.VMEM_SHARED((NTBL, F), jnp.float32),
            pltpu.SemaphoreType.DMA,
            pltpu.SemaphoreType.DMA,
            pltpu.SemaphoreType.DMA,
            pltpu.SemaphoreType.DMA,
        ],
    )
    def body(xws_hbm, src_hbm, dst_hbm, out_hbm, srcv, dstv, bufa, bufb,
             acc_s, sga, sgb, ssa, ssb):
        c = lax.axis_index("c")
        s = lax.axis_index("s")
        wid = c * 16 + s

        def zrows(i, carry):
            r = i // 8
            k = i % 8
            bufa[r, pl.ds(k * 16, 16)] = jnp.zeros((16,), jnp.float32)
            return carry

        lax.fori_loop(0, GBLK * 8, zrows, 0)

        def zacc(j, carry):
            pltpu.sync_copy(bufa, acc_s.at[pl.ds(s * 640 + j * GBLK, GBLK)])
            return carry

        lax.fori_loop(0, 640 // GBLK, zacc, 0)
        plsc.subcore_barrier()

        def grp(g, carry):
            pltpu.sync_copy(src_hbm.at[wid, pl.ds(g * 8, 8)], srcv)
            pltpu.sync_copy(dst_hbm.at[wid, pl.ds(g * 8, 8)], dstv)

            def blk2(p, carry2):
                cga = pltpu.async_copy(xws_hbm.at[srcv.at[2 * p]], bufa, sga)
                cgb = pltpu.async_copy(xws_hbm.at[srcv.at[2 * p + 1]], bufb,
                                       sgb)
                cga.wait()
                csa = pltpu.async_copy(bufa, acc_s.at[dstv.at[2 * p]], ssa,
                                       add=True)
                cgb.wait()
                csb = pltpu.async_copy(bufb, acc_s.at[dstv.at[2 * p + 1]], ssb,
                                       add=True)
                csa.wait()
                csb.wait()
                return carry2

            lax.fori_loop(0, 4, blk2, 0)
            return carry

        lax.fori_loop(0, EPT // (8 * GBLK), grp, 0)
        plsc.subcore_barrier()

        def outc(j, carry):
            off = s * 640 + j * GBLK
            pltpu.sync_copy(acc_s.at[pl.ds(off, GBLK)], bufa)
            pltpu.sync_copy(bufa, out_hbm.at[c, pl.ds(off, GBLK)])
            return carry

        lax.fori_loop(0, 640 // GBLK, outc, 0)

    return body(xws, srcp, dsts)


ABLK = 32
AGRP = 16
NGRP = EPT // (ABLK * AGRP)


def _k_att(qq, kv, srcp, dstg, dsts):
    @functools.partial(
        pl.kernel,
        out_type=(jax.ShapeDtypeStruct((2, NTBL, F), jnp.float32),
                  jax.ShapeDtypeStruct((NTILE, 2 * NTBL), jnp.float32)),
        mesh=_sc_mesh(),
        compiler_params=_SC_PARAMS,
        scratch_types=[
            pltpu.VMEM((AGRP, ABLK), jnp.int32),
            pltpu.VMEM((AGRP, ABLK), jnp.int32),
            pltpu.VMEM((AGRP, ABLK), jnp.int32),
            pltpu.VMEM((ABLK, F), jnp.float32),
            pltpu.VMEM((ABLK, 2 * F), jnp.float32),
            pltpu.VMEM((ABLK, F), jnp.float32),
            pltpu.VMEM((ABLK,), jnp.float32),
            pltpu.VMEM((ABLK,), jnp.float32),
            pltpu.VMEM((2 * NTBL,), jnp.float32),
            pltpu.VMEM_SHARED((NTBL, F), jnp.float32),
            pltpu.SemaphoreType.DMA,
            pltpu.SemaphoreType.DMA,
        ],
    )
    def body(qq_hbm, kv_hbm, src_hbm, dstg_hbm, dsts_hbm, out_hbm, den_hbm,
             srcv, dgv, dsv, qd, kvs, rows, dbm, dbl, dtab, att_s, sem1, sem2):
        c = lax.axis_index("c")
        s = lax.axis_index("s")
        wid = c * 16 + s
        zero16 = jnp.zeros((16,), jnp.float32)
        lanes = lax.iota(jnp.int32, 16)

        def zrows(i, carry):
            r = i // 8
            k = i % 8
            rows[r, pl.ds(k * 16, 16)] = zero16
            return carry

        lax.fori_loop(0, ABLK * 8, zrows, 0)

        def zden(i, carry):
            dtab[pl.ds(i * 16, 16)] = zero16
            return carry

        lax.fori_loop(0, 2 * NTBL // 16, zden, 0)

        def zacc(j, carry):
            pltpu.sync_copy(rows, att_s.at[pl.ds(s * 640 + j * ABLK, ABLK)])
            return carry

        lax.fori_loop(0, 640 // ABLK, zacc, 0)
        plsc.subcore_barrier()

        mask0 = lanes == 0

        def run_edges():
          @plsc.parallel_loop(0, ABLK, 1, unroll=8)
          def edge(e):
            amu = (qd[e, pl.ds(0, 16)] * kvs[e, pl.ds(0, 16)]
                   + qd[e, pl.ds(16, 16)] * kvs[e, pl.ds(16, 16)]
                   + qd[e, pl.ds(32, 16)] * kvs[e, pl.ds(32, 16)]
                   + qd[e, pl.ds(48, 16)] * kvs[e, pl.ds(48, 16)])
            als = (qd[e, pl.ds(64, 16)] * kvs[e, pl.ds(64, 16)]
                   + qd[e, pl.ds(80, 16)] * kvs[e, pl.ds(80, 16)]
                   + qd[e, pl.ds(96, 16)] * kvs[e, pl.ds(96, 16)]
                   + qd[e, pl.ds(112, 16)] * kvs[e, pl.ds(112, 16)])
            emu = jnp.exp(jnp.broadcast_to(jnp.sum(amu) * SCALE, (16,)))
            els = jnp.exp(jnp.broadcast_to(jnp.sum(als) * SCALE, (16,)))
            rows[e, pl.ds(0, 16)] = kvs[e, pl.ds(128, 16)] * emu
            rows[e, pl.ds(16, 16)] = kvs[e, pl.ds(144, 16)] * emu
            rows[e, pl.ds(32, 16)] = kvs[e, pl.ds(160, 16)] * emu
            rows[e, pl.ds(48, 16)] = kvs[e, pl.ds(176, 16)] * emu
            rows[e, pl.ds(64, 16)] = kvs[e, pl.ds(192, 16)] * els
            rows[e, pl.ds(80, 16)] = kvs[e, pl.ds(208, 16)] * els
            rows[e, pl.ds(96, 16)] = kvs[e, pl.ds(224, 16)] * els
            rows[e, pl.ds(112, 16)] = kvs[e, pl.ds(240, 16)] * els
            ei = jnp.full((16,), e, jnp.int32)
            plsc.store_scatter(dbm, [ei], emu, mask=mask0)
            plsc.store_scatter(dbl, [ei], els, mask=mask0)

        def make_dscat(b2):
          def dscat(t, carry):
            di = dsv[b2, pl.ds(t * 16, 16)]
            plsc.addupdate_scatter(dtab, [di * 2], dbm[pl.ds(t * 16, 16)])
            plsc.addupdate_scatter(dtab, [di * 2 + 1], dbl[pl.ds(t * 16, 16)])
            return carry
          return dscat

        def grp(g, carry):
            pltpu.sync_copy(src_hbm.at[wid, pl.ds(g * AGRP, AGRP)], srcv)
            pltpu.sync_copy(dstg_hbm.at[wid, pl.ds(g * AGRP, AGRP)], dgv)
            pltpu.sync_copy(dsts_hbm.at[wid, pl.ds(g * AGRP, AGRP)], dsv)

            def blk(b2, carry2):
                cp1 = pltpu.async_copy(qq_hbm.at[dgv.at[b2]], qd, sem1)
                cp2 = pltpu.async_copy(kv_hbm.at[srcv.at[b2]], kvs, sem2)
                cp1.wait()
                cp2.wait()
                run_edges()
                lax.fori_loop(0, ABLK // 16, make_dscat(b2), 0)
                pltpu.sync_copy(rows, att_s.at[dsv.at[b2]], add=True)
                return carry2

            lax.fori_loop(0, AGRP, blk, 0)
            return carry

        lax.fori_loop(0, NGRP, grp, 0)
        plsc.subcore_barrier()

        def outc(j, carry):
            off = s * 640 + j * ABLK
            pltpu.sync_copy(att_s.at[pl.ds(off, ABLK)], rows)
            pltpu.sync_copy(rows, out_hbm.at[c, pl.ds(off, ABLK)])
            return carry

        lax.fori_loop(0, 640 // ABLK, outc, 0)
        pltpu.sync_copy(dtab, den_hbm.at[wid])

    return body(qq, kv, srcp, dstg, dsts)


_BN = 2000


def _row_spec(w):
    return pl.BlockSpec((_BN, w), lambda i: (i, 0))


def _full_spec(h, w):
    return pl.BlockSpec((h, w), lambda i: (0, 0))


def _mm1_body(x_ref, w_ref, deg_ref, xws_ref, dis_ref):
    dis = jax.lax.rsqrt(jnp.maximum(deg_ref[...] + 1.0, 1.0))
    xw = jnp.dot(x_ref[...], w_ref[...], preferred_element_type=jnp.float32)
    xws_ref[...] = xw * dis
    dis_ref[...] = dis


def _mm1(x, w, deg):
    return pl.pallas_call(
        _mm1_body,
        grid=(N // _BN,),
        in_specs=[_row_spec(F), _full_spec(F, F), _row_spec(1)],
        out_specs=(_row_spec(F), _row_spec(1)),
        out_shape=(
            jax.ShapeDtypeStruct((N, F), jnp.float32),
            jax.ShapeDtypeStruct((N, 1), jnp.float32),
        ),
    )(x, w, deg)


def _mm2_body(accg_ref, xws_ref, dis_ref, b_ref, wcat_ref, bcat_ref,
              qq_ref, kv_ref, init_ref, dinit_ref, skip_ref):
    acc = accg_ref[0] + accg_ref[1] + xws_ref[...]
    gcn = acc * dis_ref[...] + b_ref[...]
    h = jnp.where(gcn >= 0.0, gcn, 0.01 * gcn)
    z = jnp.dot(h, wcat_ref[...], preferred_element_type=jnp.float32) + bcat_ref[...]
    q_mu, k_mu, v_mu, s_mu = z[:, 0:64], z[:, 64:128], z[:, 128:192], z[:, 192:256]
    q_ls, k_ls, v_ls, s_ls = z[:, 256:320], z[:, 320:384], z[:, 384:448], z[:, 448:512]
    qq_ref[...] = jnp.concatenate([q_mu, q_ls], axis=1)
    kv_ref[...] = jnp.concatenate([k_mu, k_ls, v_mu, v_ls], axis=1)
    e_mu = jnp.exp(jnp.sum(q_mu * k_mu, axis=1, keepdims=True) * SCALE)
    e_ls = jnp.exp(jnp.sum(q_ls * k_ls, axis=1, keepdims=True) * SCALE)
    init_ref[...] = jnp.concatenate([e_mu * v_mu, e_ls * v_ls], axis=1)
    dinit_ref[...] = jnp.concatenate([e_mu, e_ls], axis=1)
    skip_ref[...] = jnp.concatenate([s_mu, s_ls], axis=1)


def _mm2(accg, xws, dis, b_gcn, wcat, bcat):
    return pl.pallas_call(
        _mm2_body,
        grid=(N // _BN,),
        in_specs=[pl.BlockSpec((2, _BN, F), lambda i: (0, i, 0)),
                  _row_spec(F), _row_spec(1), _full_spec(1, F),
                  _full_spec(F, 512), _full_spec(1, 512)],
        out_specs=(_row_spec(2 * D), _row_spec(4 * D), _row_spec(F),
                   _row_spec(2), _row_spec(2 * D)),
        out_shape=(
            jax.ShapeDtypeStruct((N, 2 * D), jnp.float32),
            jax.ShapeDtypeStruct((N, 4 * D), jnp.float32),
            jax.ShapeDtypeStruct((N, F), jnp.float32),
            jax.ShapeDtypeStruct((N, 2), jnp.float32),
            jax.ShapeDtypeStruct((N, 2 * D), jnp.float32),
        ),
    )(accg, xws, dis, b_gcn, wcat, bcat)


def _dred2_body(d_ref, out_ref):
    out_ref[...] = jnp.sum(d_ref[...], axis=0)


def _dred2(den32):
    return pl.pallas_call(
        _dred2_body,
        grid=(8,),
        in_specs=[pl.BlockSpec((NTILE, NTBL // 8, 2), lambda i: (0, i, 0))],
        out_specs=pl.BlockSpec((NTBL // 8, 2), lambda i: (i, 0)),
        out_shape=jax.ShapeDtypeStruct((NTBL, 2), jnp.float32),
    )(den32.reshape(NTILE, NTBL, 2))


def _out_body(attg_ref, den_ref, init_ref, dinit_ref, skip_ref, mu_ref, ls_ref):
    att = attg_ref[0] + attg_ref[1] + init_ref[...]
    den = den_ref[...] + dinit_ref[...]
    mu = att[:, 0:64] / (den[:, 0:1] + 1e-16) + skip_ref[:, 0:64]
    ls = att[:, 64:128] / (den[:, 1:2] + 1e-16) + skip_ref[:, 64:128]
    mu_ref[...] = mu
    ls_ref[...] = jnp.minimum(ls, 10.0)


def _out(attg, den, init, dinit, skip):
    return pl.pallas_call(
        _out_body,
        grid=(N // _BN,),
        in_specs=[pl.BlockSpec((2, _BN, F), lambda i: (0, i, 0)),
                  _row_spec(2), _row_spec(F), _row_spec(2), _row_spec(2 * D)],
        out_specs=(_row_spec(D), _row_spec(D)),
        out_shape=(
            jax.ShapeDtypeStruct((N, D), jnp.float32),
            jax.ShapeDtypeStruct((N, D), jnp.float32),
        ),
    )(attg, den, init, dinit, skip)


def kernel(x, edge_index, W_gcn, b_gcn, Wq_mu, bq_mu, Wk_mu, bk_mu, Wv_mu, bv_mu,
           Ws_mu, bs_mu, Wq_ls, bq_ls, Wk_ls, bk_ls, Wv_ls, bv_ls, Ws_ls, bs_ls):
    src = edge_index[0]
    dst = edge_index[1]

    pad = EP - E
    srcp = jnp.concatenate([src, jnp.zeros((pad,), jnp.int32)])
    srcp = srcp.reshape(NTILE, NBLK, BLK)
    dsts = jnp.concatenate([dst, jnp.full((pad,), N, jnp.int32)])
    dsts = dsts.reshape(NTILE, NBLK, BLK)
    dstg = jnp.concatenate([dst, jnp.zeros((pad,), jnp.int32)])
    dstg = dstg.reshape(NTILE, NBLK, BLK)

    deg = _dred(_k_deg(dsts))
    xws, dis = _mm1(x, W_gcn, deg[:N])

    gshape = (NTILE, EPT // GBLK, GBLK)
    accg = _k_gcn(xws, srcp.reshape(gshape), dsts.reshape(gshape))

    wcat = jnp.concatenate(
        [Wq_mu, Wk_mu, Wv_mu, Ws_mu, Wq_ls, Wk_ls, Wv_ls, Ws_ls], axis=1)
    bcat = jnp.concatenate(
        [bq_mu, bk_mu, bv_mu, bs_mu, bq_ls, bk_ls, bv_ls, bs_ls])[None, :]
    qq, kv, att_init, den_init, skip = _mm2(accg, xws, dis, b_gcn[None, :],
                                            wcat, bcat)

    ashape = (NTILE, EPT // ABLK, ABLK)
    attg, den32 = _k_att(qq, kv, srcp.reshape(ashape), dstg.reshape(ashape),
                         dsts.reshape(ashape))
    den = _dred2(den32)

    return _out(attg, den[:N], att_init, den_init, skip)


# speedup vs baseline: 1.0850x; 1.0251x over previous
import functools

import jax
import jax.numpy as jnp
import numpy as np
from jax import lax
from jax.experimental import pallas as pl
from jax.experimental.pallas import tpu as pltpu
from jax.experimental.pallas import tpu_sc as plsc

N = 10000
E = 320000
F = 128
D = 64
SCALE = 0.125
ACC_W = 144

NTILE = 32
EPT = 10240
EP = NTILE * EPT
NBLK = 80
BLK = 128
NTBL = 10240


def _sc_mesh():
    return plsc.VectorSubcoreMesh(core_axis_name="c", subcore_axis_name="s")


_SC_PARAMS = pltpu.CompilerParams(needs_layout_passes=False)


def _k_deg(dsts):
    @functools.partial(
        pl.kernel,
        out_type=jax.ShapeDtypeStruct((NTILE, NTBL), jnp.float32),
        mesh=_sc_mesh(),
        compiler_params=_SC_PARAMS,
        scratch_types=[
            pltpu.VMEM((NBLK, BLK), jnp.int32),
            pltpu.VMEM((NTBL,), jnp.float32),
        ],
    )
    def body(dst_hbm, out_hbm, dstv, degv):
        c = lax.axis_index("c")
        s = lax.axis_index("s")
        wid = c * 16 + s
        pltpu.sync_copy(dst_hbm.at[wid], dstv)

        def zero(i, carry):
            degv[pl.ds(i * 16, 16)] = jnp.zeros((16,), jnp.float32)
            return carry

        lax.fori_loop(0, NTBL // 16, zero, 0)
        ones = jnp.ones((16,), jnp.float32)

        def scat(i, carry):
            b = i // 8
            k = i % 8
            idx = dstv[b, pl.ds(k * 16, 16)]
            plsc.addupdate_scatter(degv, [idx], ones)
            return carry

        lax.fori_loop(0, EPT // 16, scat, 0)
        pltpu.sync_copy(degv, out_hbm.at[wid])

    return body(dsts)


def _dred_body(d_ref, out_ref):
    out_ref[...] = jnp.sum(d_ref[...], axis=0)[:, None]


def _dred(deg32):
    return pl.pallas_call(
        _dred_body,
        grid=(8,),
        in_specs=[pl.BlockSpec((NTILE, NTBL // 8), lambda i: (0, i))],
        out_specs=pl.BlockSpec((NTBL // 8, 1), lambda i: (i, 0)),
        out_shape=jax.ShapeDtypeStruct((NTBL, 1), jnp.float32),
    )(deg32)


GBLK = 128


def _k_gcn(xws, srcp, dsts):
    @functools.partial(
        pl.kernel,
        out_type=jax.ShapeDtypeStruct((2, NTBL, F), jnp.float32),
        mesh=_sc_mesh(),
        compiler_params=_SC_PARAMS,
        scratch_types=[
            pltpu.VMEM((8, GBLK), jnp.int32),
            pltpu.VMEM((8, GBLK), jnp.int32),
            pltpu.VMEM((GBLK, F), jnp.float32),
            pltpu.VMEM((GBLK, F), jnp.float32),
            pltpu.VMEM_SHARED((NTBL, F), jnp.float32),
            pltpu.SemaphoreType.DMA,
            pltpu.SemaphoreType.DMA,
            pltpu.SemaphoreType.DMA,
            pltpu.SemaphoreType.DMA,
        ],
    )
    def body(xws_hbm, src_hbm, dst_hbm, out_hbm, srcv, dstv, bufa, bufb,
             acc_s, sga, sgb, ssa, ssb):
        c = lax.axis_index("c")
        s = lax.axis_index("s")
        wid = c * 16 + s

        def zrows(i, carry):
            r = i // 8
            k = i % 8
            bufa[r, pl.ds(k * 16, 16)] = jnp.zeros((16,), jnp.float32)
            return carry

        lax.fori_loop(0, GBLK * 8, zrows, 0)

        def zacc(j, carry):
            pltpu.sync_copy(bufa, acc_s.at[pl.ds(s * 640 + j * GBLK, GBLK)])
            return carry

        lax.fori_loop(0, 640 // GBLK, zacc, 0)
        plsc.subcore_barrier()

        def grp(g, carry):
            pltpu.sync_copy(src_hbm.at[wid, pl.ds(g * 8, 8)], srcv)
            pltpu.sync_copy(dst_hbm.at[wid, pl.ds(g * 8, 8)], dstv)

            def blk2(p, carry2):
                cga = pltpu.async_copy(xws_hbm.at[srcv.at[2 * p]], bufa, sga)
                cgb = pltpu.async_copy(xws_hbm.at[srcv.at[2 * p + 1]], bufb,
                                       sgb)
                cga.wait()
                csa = pltpu.async_copy(bufa, acc_s.at[dstv.at[2 * p]], ssa,
                                       add=True)
                cgb.wait()
                csb = pltpu.async_copy(bufb, acc_s.at[dstv.at[2 * p + 1]], ssb,
                                       add=True)
                csa.wait()
                csb.wait()
                return carry2

            lax.fori_loop(0, 4, blk2, 0)
            return carry

        lax.fori_loop(0, EPT // (8 * GBLK), grp, 0)
        plsc.subcore_barrier()

        def outc(j, carry):
            off = s * 640 + j * GBLK
            pltpu.sync_copy(acc_s.at[pl.ds(off, GBLK)], bufa)
            pltpu.sync_copy(bufa, out_hbm.at[c, pl.ds(off, GBLK)])
            return carry

        lax.fori_loop(0, 640 // GBLK, outc, 0)

    return body(xws, srcp, dsts)


ABLK = 32
AGRP = 16
NGRP = EPT // (ABLK * AGRP)


def _k_att(qq, kv, srcp, dstg, dsts):
    @functools.partial(
        pl.kernel,
        out_type=(jax.ShapeDtypeStruct((2, NTBL, F), jnp.float32),
                  jax.ShapeDtypeStruct((NTILE, 2 * NTBL), jnp.float32)),
        mesh=_sc_mesh(),
        compiler_params=_SC_PARAMS,
        scratch_types=[
            pltpu.VMEM((AGRP, ABLK), jnp.int32),
            pltpu.VMEM((AGRP, ABLK), jnp.int32),
            pltpu.VMEM((AGRP, ABLK), jnp.int32),
            pltpu.VMEM((ABLK, F), jnp.float32),
            pltpu.VMEM((ABLK, 2 * F), jnp.float32),
            pltpu.VMEM((ABLK, F), jnp.float32),
            pltpu.VMEM((ABLK, F), jnp.float32),
            pltpu.VMEM((ABLK,), jnp.float32),
            pltpu.VMEM((ABLK,), jnp.float32),
            pltpu.VMEM((2 * NTBL,), jnp.float32),
            pltpu.VMEM_SHARED((NTBL, F), jnp.float32),
            pltpu.SemaphoreType.DMA,
            pltpu.SemaphoreType.DMA,
            pltpu.SemaphoreType.DMA,
            pltpu.SemaphoreType.DMA,
        ],
    )
    def body(qq_hbm, kv_hbm, src_hbm, dstg_hbm, dsts_hbm, out_hbm, den_hbm,
             srcv, dgv, dsv, qd, kvs, rowsa, rowsb, dbm, dbl, dtab, att_s,
             sem1, sem2, sema, semb):
        c = lax.axis_index("c")
        s = lax.axis_index("s")
        wid = c * 16 + s
        zero16 = jnp.zeros((16,), jnp.float32)
        lanes = lax.iota(jnp.int32, 16)

        def zrows(i, carry):
            r = i // 8
            k = i % 8
            rowsa[r, pl.ds(k * 16, 16)] = zero16
            return carry

        lax.fori_loop(0, ABLK * 8, zrows, 0)

        def zden(i, carry):
            dtab[pl.ds(i * 16, 16)] = zero16
            return carry

        lax.fori_loop(0, 2 * NTBL // 16, zden, 0)

        def zacc(j, carry):
            pltpu.sync_copy(rowsa, att_s.at[pl.ds(s * 640 + j * ABLK, ABLK)])
            return carry

        lax.fori_loop(0, 640 // ABLK, zacc, 0)
        plsc.subcore_barrier()

        mask0 = lanes == 0

        def run_edges(rows):
          @plsc.parallel_loop(0, ABLK, 1, unroll=8)
          def edge(e):
            amu = (qd[e, pl.ds(0, 16)] * kvs[e, pl.ds(0, 16)]
                   + qd[e, pl.ds(16, 16)] * kvs[e, pl.ds(16, 16)]
                   + qd[e, pl.ds(32, 16)] * kvs[e, pl.ds(32, 16)]
                   + qd[e, pl.ds(48, 16)] * kvs[e, pl.ds(48, 16)])
            als = (qd[e, pl.ds(64, 16)] * kvs[e, pl.ds(64, 16)]
                   + qd[e, pl.ds(80, 16)] * kvs[e, pl.ds(80, 16)]
                   + qd[e, pl.ds(96, 16)] * kvs[e, pl.ds(96, 16)]
                   + qd[e, pl.ds(112, 16)] * kvs[e, pl.ds(112, 16)])
            emu = jnp.exp(jnp.broadcast_to(jnp.sum(amu) * SCALE, (16,)))
            els = jnp.exp(jnp.broadcast_to(jnp.sum(als) * SCALE, (16,)))
            rows[e, pl.ds(0, 16)] = kvs[e, pl.ds(128, 16)] * emu
            rows[e, pl.ds(16, 16)] = kvs[e, pl.ds(144, 16)] * emu
            rows[e, pl.ds(32, 16)] = kvs[e, pl.ds(160, 16)] * emu
            rows[e, pl.ds(48, 16)] = kvs[e, pl.ds(176, 16)] * emu
            rows[e, pl.ds(64, 16)] = kvs[e, pl.ds(192, 16)] * els
            rows[e, pl.ds(80, 16)] = kvs[e, pl.ds(208, 16)] * els
            rows[e, pl.ds(96, 16)] = kvs[e, pl.ds(224, 16)] * els
            rows[e, pl.ds(112, 16)] = kvs[e, pl.ds(240, 16)] * els
            ei = jnp.full((16,), e, jnp.int32)
            plsc.store_scatter(dbm, [ei], emu, mask=mask0)
            plsc.store_scatter(dbl, [ei], els, mask=mask0)

        def make_dscat(b2):
          def dscat(t, carry):
            di = dsv[b2, pl.ds(t * 16, 16)]
            plsc.addupdate_scatter(dtab, [di * 2], dbm[pl.ds(t * 16, 16)])
            plsc.addupdate_scatter(dtab, [di * 2 + 1], dbl[pl.ds(t * 16, 16)])
            return carry
          return dscat

        def grp(g, carry):
            pltpu.sync_copy(src_hbm.at[wid, pl.ds(g * AGRP, AGRP)], srcv)
            pltpu.sync_copy(dstg_hbm.at[wid, pl.ds(g * AGRP, AGRP)], dgv)
            pltpu.sync_copy(dsts_hbm.at[wid, pl.ds(g * AGRP, AGRP)], dsv)

            def pair(p, carry2):
                ba = 2 * p
                bb = 2 * p + 1
                cp1 = pltpu.async_copy(qq_hbm.at[dgv.at[ba]], qd, sem1)
                cp2 = pltpu.async_copy(kv_hbm.at[srcv.at[ba]], kvs, sem2)
                cp1.wait()
                cp2.wait()
                run_edges(rowsa)
                cp3 = pltpu.async_copy(qq_hbm.at[dgv.at[bb]], qd, sem1)
                cp4 = pltpu.async_copy(kv_hbm.at[srcv.at[bb]], kvs, sem2)
                lax.fori_loop(0, ABLK // 16, make_dscat(ba), 0)
                csa = pltpu.async_copy(rowsa, att_s.at[dsv.at[ba]], sema,
                                       add=True)
                cp3.wait()
                cp4.wait()
                run_edges(rowsb)
                lax.fori_loop(0, ABLK // 16, make_dscat(bb), 0)
                csb = pltpu.async_copy(rowsb, att_s.at[dsv.at[bb]], semb,
                                       add=True)
                csa.wait()
                csb.wait()
                return carry2

            lax.fori_loop(0, AGRP // 2, pair, 0)
            return carry

        lax.fori_loop(0, NGRP, grp, 0)
        plsc.subcore_barrier()

        def outc(j, carry):
            off = s * 640 + j * ABLK
            pltpu.sync_copy(att_s.at[pl.ds(off, ABLK)], rowsa)
            pltpu.sync_copy(rowsa, out_hbm.at[c, pl.ds(off, ABLK)])
            return carry

        lax.fori_loop(0, 640 // ABLK, outc, 0)
        pltpu.sync_copy(dtab, den_hbm.at[wid])

    return body(qq, kv, srcp, dstg, dsts)


_BN = 2000


def _row_spec(w):
    return pl.BlockSpec((_BN, w), lambda i: (i, 0))


def _full_spec(h, w):
    return pl.BlockSpec((h, w), lambda i: (0, 0))


def _mm1_body(x_ref, w_ref, deg_ref, xws_ref, dis_ref):
    dis = jax.lax.rsqrt(jnp.maximum(deg_ref[...] + 1.0, 1.0))
    xw = jnp.dot(x_ref[...], w_ref[...], preferred_element_type=jnp.float32)
    xws_ref[...] = xw * dis
    dis_ref[...] = dis


def _mm1(x, w, deg):
    return pl.pallas_call(
        _mm1_body,
        grid=(N // _BN,),
        in_specs=[_row_spec(F), _full_spec(F, F), _row_spec(1)],
        out_specs=(_row_spec(F), _row_spec(1)),
        out_shape=(
            jax.ShapeDtypeStruct((N, F), jnp.float32),
            jax.ShapeDtypeStruct((N, 1), jnp.float32),
        ),
    )(x, w, deg)


def _mm2_body(accg_ref, xws_ref, dis_ref, b_ref, wcat_ref, bcat_ref,
              qq_ref, kv_ref, init_ref, dinit_ref, skip_ref):
    acc = accg_ref[0] + accg_ref[1] + xws_ref[...]
    gcn = acc * dis_ref[...] + b_ref[...]
    h = jnp.where(gcn >= 0.0, gcn, 0.01 * gcn)
    z = jnp.dot(h, wcat_ref[...], preferred_element_type=jnp.float32) + bcat_ref[...]
    q_mu, k_mu, v_mu, s_mu = z[:, 0:64], z[:, 64:128], z[:, 128:192], z[:, 192:256]
    q_ls, k_ls, v_ls, s_ls = z[:, 256:320], z[:, 320:384], z[:, 384:448], z[:, 448:512]
    qq_ref[...] = jnp.concatenate([q_mu, q_ls], axis=1)
    kv_ref[...] = jnp.concatenate([k_mu, k_ls, v_mu, v_ls], axis=1)
    e_mu = jnp.exp(jnp.sum(q_mu * k_mu, axis=1, keepdims=True) * SCALE)
    e_ls = jnp.exp(jnp.sum(q_ls * k_ls, axis=1, keepdims=True) * SCALE)
    init_ref[...] = jnp.concatenate([e_mu * v_mu, e_ls * v_ls], axis=1)
    dinit_ref[...] = jnp.concatenate([e_mu, e_ls], axis=1)
    skip_ref[...] = jnp.concatenate([s_mu, s_ls], axis=1)


def _mm2(accg, xws, dis, b_gcn, wcat, bcat):
    return pl.pallas_call(
        _mm2_body,
        grid=(N // _BN,),
        in_specs=[pl.BlockSpec((2, _BN, F), lambda i: (0, i, 0)),
                  _row_spec(F), _row_spec(1), _full_spec(1, F),
                  _full_spec(F, 512), _full_spec(1, 512)],
        out_specs=(_row_spec(2 * D), _row_spec(4 * D), _row_spec(F),
                   _row_spec(2), _row_spec(2 * D)),
        out_shape=(
            jax.ShapeDtypeStruct((N, 2 * D), jnp.float32),
            jax.ShapeDtypeStruct((N, 4 * D), jnp.float32),
            jax.ShapeDtypeStruct((N, F), jnp.float32),
            jax.ShapeDtypeStruct((N, 2), jnp.float32),
            jax.ShapeDtypeStruct((N, 2 * D), jnp.float32),
        ),
    )(accg, xws, dis, b_gcn, wcat, bcat)


def _dred2_body(d_ref, out_ref):
    out_ref[...] = jnp.sum(d_ref[...], axis=0)


def _dred2(den32):
    return pl.pallas_call(
        _dred2_body,
        grid=(8,),
        in_specs=[pl.BlockSpec((NTILE, NTBL // 8, 2), lambda i: (0, i, 0))],
        out_specs=pl.BlockSpec((NTBL // 8, 2), lambda i: (i, 0)),
        out_shape=jax.ShapeDtypeStruct((NTBL, 2), jnp.float32),
    )(den32.reshape(NTILE, NTBL, 2))


def _out_body(attg_ref, den_ref, init_ref, dinit_ref, skip_ref, mu_ref, ls_ref):
    att = attg_ref[0] + attg_ref[1] + init_ref[...]
    den = den_ref[...] + dinit_ref[...]
    mu = att[:, 0:64] / (den[:, 0:1] + 1e-16) + skip_ref[:, 0:64]
    ls = att[:, 64:128] / (den[:, 1:2] + 1e-16) + skip_ref[:, 64:128]
    mu_ref[...] = mu
    ls_ref[...] = jnp.minimum(ls, 10.0)


def _out(attg, den, init, dinit, skip):
    return pl.pallas_call(
        _out_body,
        grid=(N // _BN,),
        in_specs=[pl.BlockSpec((2, _BN, F), lambda i: (0, i, 0)),
                  _row_spec(2), _row_spec(F), _row_spec(2), _row_spec(2 * D)],
        out_specs=(_row_spec(D), _row_spec(D)),
        out_shape=(
            jax.ShapeDtypeStruct((N, D), jnp.float32),
            jax.ShapeDtypeStruct((N, D), jnp.float32),
        ),
    )(attg, den, init, dinit, skip)


def kernel(x, edge_index, W_gcn, b_gcn, Wq_mu, bq_mu, Wk_mu, bk_mu, Wv_mu, bv_mu,
           Ws_mu, bs_mu, Wq_ls, bq_ls, Wk_ls, bk_ls, Wv_ls, bv_ls, Ws_ls, bs_ls):
    src = edge_index[0]
    dst = edge_index[1]

    pad = EP - E
    srcp = jnp.concatenate([src, jnp.zeros((pad,), jnp.int32)])
    srcp = srcp.reshape(NTILE, NBLK, BLK)
    dsts = jnp.concatenate([dst, jnp.full((pad,), N, jnp.int32)])
    dsts = dsts.reshape(NTILE, NBLK, BLK)
    dstg = jnp.concatenate([dst, jnp.zeros((pad,), jnp.int32)])
    dstg = dstg.reshape(NTILE, NBLK, BLK)

    deg = _dred(_k_deg(dsts))
    xws, dis = _mm1(x, W_gcn, deg[:N])

    gshape = (NTILE, EPT // GBLK, GBLK)
    accg = _k_gcn(xws, srcp.reshape(gshape), dsts.reshape(gshape))

    wcat = jnp.concatenate(
        [Wq_mu, Wk_mu, Wv_mu, Ws_mu, Wq_ls, Wk_ls, Wv_ls, Ws_ls], axis=1)
    bcat = jnp.concatenate(
        [bq_mu, bk_mu, bv_mu, bs_mu, bq_ls, bk_ls, bv_ls, bs_ls])[None, :]
    qq, kv, att_init, den_init, skip = _mm2(accg, xws, dis, b_gcn[None, :],
                                            wcat, bcat)

    ashape = (NTILE, EPT // ABLK, ABLK)
    attg, den32 = _k_att(qq, kv, srcp.reshape(ashape), dstg.reshape(ashape),
                         dsts.reshape(ashape))
    den = _dred2(den32)

    return _out(attg, den[:N], att_init, den_init, skip)
